# Initial kernel scaffold; baseline (speedup 1.0000x reference)
import jax
import jax.numpy as jnp
from jax.experimental import pallas as pl


def kernel(x, y, weights):
    raise NotImplementedError("write your pallas kernel here")


# speedup vs baseline: 1.9159x; 1.9159x over previous
import numpy as np
import jax
import jax.numpy as jnp
from jax.experimental import pallas as pl

_INV_SQRT3 = float(1.0 / np.sqrt(3.0))


def _sel_matrix(C: int) -> np.ndarray:
    m = np.zeros((4 * C, 2 * C), np.float32)
    for c in range(C):
        m[4 * c, c] = 1.0
        for d in range(4):
            m[4 * c + d, C + c] = 1.0
    return m


def _tc_body(x_ref, y_ref, m_ref, w_ref, o_ref):
    xf = x_ref[...]
    yf = y_ref[...]
    p = xf * yf
    m = m_ref[...]
    xs = jnp.dot(xf, m, preferred_element_type=jnp.float32)
    ys = jnp.dot(yf, m, preferred_element_type=jnp.float32)
    ps = jnp.dot(p, m, preferred_element_type=jnp.float32)
    C = o_ref.shape[1]
    x0, sx = xs[:, :C], xs[:, C:]
    y0, sy = ys[:, :C], ys[:, C:]
    p0, g = ps[:, :C], ps[:, C:]
    w = w_ref[...]
    o_ref[...] = (w[0:1] * p0 + w[1:2] * (x0 * (sy - y0))
                  + w[2:3] * ((sx - x0) * y0) + w[3:4] * (g - p0))


def kernel(x, y, weights):
    B, C, D = x.shape
    xf = x.reshape(B, C * D)
    yf = y.reshape(B, C * D)
    wmat = jnp.concatenate(
        [weights[:, 0:1], weights[:, 1:2], weights[:, 2:3],
         weights[:, 3:4] * _INV_SQRT3,
         jnp.zeros((C, 4), jnp.float32)], axis=1).T
    bs = 1600
    return pl.pallas_call(
        _tc_body,
        grid=(B // bs,),
        in_specs=[
            pl.BlockSpec((bs, C * D), lambda i: (i, 0)),
            pl.BlockSpec((bs, C * D), lambda i: (i, 0)),
            pl.BlockSpec((C * D, 2 * C), lambda i: (0, 0)),
            pl.BlockSpec((8, C), lambda i: (0, 0)),
        ],
        out_specs=pl.BlockSpec((bs, C), lambda i: (i, 0)),
        out_shape=jax.ShapeDtypeStruct((B, C), jnp.float32),
    )(xf, yf, jnp.asarray(_sel_matrix(C)), wmat)
